# Initial kernel scaffold; baseline (speedup 1.0000x reference)
import jax
import jax.numpy as jnp
from jax.experimental import pallas as pl


def kernel(x, edge_index, edge_weight, W1, b1, W2, b2, Wl1, bl1, Wl2, bl2):
    raise NotImplementedError("write your pallas kernel here")


# speedup vs baseline: 2.8061x; 2.8061x over previous
import functools

import jax
import jax.numpy as jnp
from jax import lax
from jax.experimental import pallas as pl
from jax.experimental.pallas import tpu as pltpu
from jax.experimental.pallas import tpu_sc as plsc

N = 10000
E = 320000
D = 128
NPAD = 10240
NTILES = 32
CH = 128
CHUNKS = 80
EPT = CH * CHUNKS
E_PAD = NTILES * EPT
ROWS_PER_TILE = NPAD // 16

_PREC = lax.Precision.HIGHEST


def _sc_conv_body(h_hbm, src_hbm, dst_hbm, w_hbm, out_hbm,
                  src_v, dst_v, w_v, rows, acc, sem):
    cid = lax.axis_index("c")
    sid = lax.axis_index("s")
    tid = cid * 16 + sid

    pltpu.sync_copy(src_hbm.at[tid], src_v)
    pltpu.sync_copy(dst_hbm.at[tid], dst_v)
    pltpu.sync_copy(w_hbm.at[tid], w_v)

    def _zero_body(i, _):
        for v in range(D // 16):
            rows[i, pl.ds(v * 16, 16)] = jnp.zeros((16,), jnp.float32)
        return 0
    lax.fori_loop(0, CH, _zero_body, 0)
    base = sid * ROWS_PER_TILE
    for k in range(ROWS_PER_TILE // CH):
        pltpu.sync_copy(rows, acc.at[pl.ds(base + k * CH, CH)])
    plsc.subcore_barrier()

    def _chunk_body(j, _):
        pltpu.async_copy(h_hbm.at[src_v.at[j]], rows, sem).wait()

        def _grp_body(g, _c):
            wv = w_v[j, pl.ds(g * 16, 16)]
            for l in range(16):
                e = g * 16 + l
                w = wv[l]
                for v in range(D // 16):
                    sl = pl.ds(v * 16, 16)
                    rows[e, sl] = rows[e, sl] * w
            return 0
        lax.fori_loop(0, CH // 16, _grp_body, 0)

        pltpu.sync_copy(rows, acc.at[dst_v.at[j]], add=True)
        return 0
    lax.fori_loop(0, CHUNKS, _chunk_body, 0)

    plsc.subcore_barrier()
    pltpu.sync_copy(acc.at[pl.ds(base, ROWS_PER_TILE)],
                    out_hbm.at[cid, pl.ds(base, ROWS_PER_TILE)])


@functools.cache
def _sc_conv():
    mesh = plsc.VectorSubcoreMesh(core_axis_name="c", subcore_axis_name="s")
    return pl.kernel(
        _sc_conv_body,
        mesh=mesh,
        out_type=jax.ShapeDtypeStruct((2, NPAD, D), jnp.float32),
        scratch_types=[
            pltpu.VMEM((CHUNKS, CH), jnp.int32),
            pltpu.VMEM((CHUNKS, CH), jnp.int32),
            pltpu.VMEM((CHUNKS, CH), jnp.float32),
            pltpu.VMEM((CH, D), jnp.float32),
            pltpu.VMEM_SHARED((NPAD, D), jnp.float32),
            pltpu.SemaphoreType.DMA,
        ],
    )


_BLK = 1000
_GRID = N // _BLK


def _mm1_body(x_ref, w_ref, o_ref):
    o_ref[...] = jnp.dot(x_ref[...], w_ref[...],
                         preferred_element_type=jnp.float32, precision=_PREC)


def _mm1(x, w):
    return pl.pallas_call(
        _mm1_body,
        grid=(_GRID,),
        in_specs=[pl.BlockSpec((_BLK, D), lambda i: (i, 0)),
                  pl.BlockSpec((D, D), lambda i: (0, 0))],
        out_specs=pl.BlockSpec((_BLK, D), lambda i: (i, 0)),
        out_shape=jax.ShapeDtypeStruct((N, D), jnp.float32),
    )(x, w)


def _mm2_body(p0_ref, p1_ref, b_ref, w_ref, o_ref):
    h = p0_ref[...] + p1_ref[...] + b_ref[...]
    o_ref[...] = jnp.dot(h, w_ref[...],
                         preferred_element_type=jnp.float32, precision=_PREC)


def _mm2(p0, p1, b, w):
    return pl.pallas_call(
        _mm2_body,
        grid=(_GRID,),
        in_specs=[pl.BlockSpec((_BLK, D), lambda i: (i, 0)),
                  pl.BlockSpec((_BLK, D), lambda i: (i, 0)),
                  pl.BlockSpec((D,), lambda i: (0,)),
                  pl.BlockSpec((D, D), lambda i: (0, 0))],
        out_specs=pl.BlockSpec((_BLK, D), lambda i: (i, 0)),
        out_shape=jax.ShapeDtypeStruct((N, D), jnp.float32),
    )(p0, p1, b, w)


def _final_body(p0_ref, p1_ref, b2_ref, wl1_ref, bl1_ref, wl2_ref, bl2_ref,
                o_ref):
    h = p0_ref[...] + p1_ref[...] + b2_ref[...]
    u = jnp.dot(h, wl1_ref[...], preferred_element_type=jnp.float32,
                precision=_PREC) + bl1_ref[...]
    t = jnp.dot(u, wl2_ref[...], preferred_element_type=jnp.float32,
                precision=_PREC) + bl2_ref[...]
    t = t - jnp.max(t, axis=-1, keepdims=True)
    et = jnp.exp(t)
    o_ref[...] = et / jnp.sum(et, axis=-1, keepdims=True)


def _final(p0, p1, b2, wl1, bl1, wl2, bl2):
    dh = D // 2
    do = 64
    return pl.pallas_call(
        _final_body,
        grid=(_GRID,),
        in_specs=[pl.BlockSpec((_BLK, D), lambda i: (i, 0)),
                  pl.BlockSpec((_BLK, D), lambda i: (i, 0)),
                  pl.BlockSpec((D,), lambda i: (0,)),
                  pl.BlockSpec((D, dh), lambda i: (0, 0)),
                  pl.BlockSpec((dh,), lambda i: (0,)),
                  pl.BlockSpec((dh, do), lambda i: (0, 0)),
                  pl.BlockSpec((do,), lambda i: (0,)),
                  ],
        out_specs=pl.BlockSpec((_BLK, do), lambda i: (i, 0)),
        out_shape=jax.ShapeDtypeStruct((N, do), jnp.float32),
    )(p0, p1, b2, wl1, bl1, wl2, bl2)


def kernel(x, edge_index, edge_weight, W1, b1, W2, b2, Wl1, bl1, Wl2, bl2):
    src = edge_index[0].astype(jnp.int32)
    dst = edge_index[1].astype(jnp.int32)
    w = edge_weight.astype(jnp.float32)

    pad = E_PAD - E
    src_p = jnp.concatenate([src, jnp.zeros((pad,), jnp.int32)])
    dst_p = jnp.concatenate([dst, jnp.zeros((pad,), jnp.int32)])
    w_p = jnp.concatenate([w, jnp.zeros((pad,), jnp.float32)])
    src_p = src_p.reshape(NTILES, CHUNKS, CH)
    dst_p = dst_p.reshape(NTILES, CHUNKS, CH)
    w_p = w_p.reshape(NTILES, CHUNKS, CH)

    sc_conv = _sc_conv()
    h0 = _mm1(x, W1)
    part1 = sc_conv(h0, src_p, dst_p, w_p)
    h1 = _mm2(part1[0, :N], part1[1, :N], b1, W2)
    part2 = sc_conv(h1, src_p, dst_p, w_p)
    return _final(part2[0, :N], part2[1, :N], b2, Wl1, bl1, Wl2, bl2)
